# baseline (speedup 1.0000x reference)
import jax
import jax.numpy as jnp
from jax import lax
from jax.experimental import pallas as pl
from jax.experimental.pallas import tpu as pltpu, tpu_sc as plsc

N = 10000
E = 320000
C = 128
NEG_SLOPE = 0.2

NW = 32
EPW = E // NW
CH = 80
NCH = EPW // CH
W1 = 25
NWIN1 = NCH // W1
W2 = 20
NWIN2 = 6
TAIL2 = NCH - NWIN2 * W2
TSL = 1000

_SC_PARAMS = pltpu.CompilerParams(
    use_tc_tiling_on_sc=False, needs_layout_passes=False)


def _stage_a_body(x_ref, w_ref, att2_ref, xw16_ref, s2_ref):
    xwb = jnp.dot(x_ref[...], w_ref[...], preferred_element_type=jnp.float32)
    xw16_ref[...] = xwb.astype(jnp.bfloat16)
    s2_ref[...] = lax.dot_general(
        att2_ref[...], xwb, (((1,), (1,)), ((), ())),
        preferred_element_type=jnp.float32)


def _stage_a(x, weight, att2):
    return pl.pallas_call(
        _stage_a_body,
        out_shape=[
            jax.ShapeDtypeStruct((N, C), jnp.bfloat16),
            jax.ShapeDtypeStruct((2, N), jnp.float32),
        ],
    )(x, weight, att2)


def _pass1_body(s2_hbm, ei4_hbm, zeros_hbm, den_part_hbm, w_hbm,
                src_idx_v, dst_idx_v, sdst_v, ssrc_v, w8a_v, w8b_v, wlin_v,
                den_sh, wsema, wsemb):
    core = lax.axis_index("c")
    sub = lax.axis_index("s")
    wid = core * 16 + sub

    pltpu.sync_copy(s2_hbm.at[0], sdst_v)
    pltpu.sync_copy(s2_hbm.at[1], ssrc_v)
    pltpu.sync_copy(zeros_hbm.at[pl.ds(0, CH), pl.ds(0, 8)], w8a_v)
    pltpu.sync_copy(zeros_hbm.at[pl.ds(0, CH), pl.ds(0, 8)], w8b_v)

    @pl.when(sub < 10)
    def _zero():
        pltpu.sync_copy(zeros_hbm.at[pl.ds(TSL * sub, TSL), pl.ds(0, 8)],
                        den_sh.at[pl.ds(TSL * sub, TSL)])
    plsc.subcore_barrier()

    zeros16 = jnp.zeros((16,), jnp.int32)
    iota16 = lax.iota(jnp.int32, 16)

    def compute_w(j, w8):
        for k in range(CH // 16):
            di = dst_idx_v[j, pl.ds(16 * k, 16)]
            si = src_idx_v[j, pl.ds(16 * k, 16)]
            a = (plsc.load_gather(sdst_v, [di])
                 + plsc.load_gather(ssrc_v, [si]))
            a = jnp.maximum(a, NEG_SLOPE * a)
            w = jnp.exp(a)
            plsc.store_scatter(w8, [iota16 + 16 * k, zeros16], w)
            wlin_v[pl.ds(CH * j + 16 * k, 16)] = w

    def window_body(win, carry0):
        pltpu.sync_copy(ei4_hbm.at[0, wid, pl.ds(W1 * win, W1)], src_idx_v)
        pltpu.sync_copy(ei4_hbm.at[1, wid, pl.ds(W1 * win, W1)], dst_idx_v)

        def pair_body(jp, carry):
            ja = 2 * jp
            jb = 2 * jp + 1
            compute_w(ja, w8a_v)
            wsa = pltpu.async_copy(w8a_v, den_sh.at[dst_idx_v.at[ja]], wsema,
                                   add=True)
            compute_w(jb, w8b_v)
            wsb = pltpu.async_copy(w8b_v, den_sh.at[dst_idx_v.at[jb]], wsemb,
                                   add=True)
            wsa.wait()
            wsb.wait()
            return carry

        lax.fori_loop(0, W1 // 2, pair_body, 0)
        compute_w(W1 - 1, w8a_v)
        pltpu.async_copy(w8a_v, den_sh.at[dst_idx_v.at[W1 - 1]], wsema,
                         add=True).wait()
        pltpu.sync_copy(wlin_v, w_hbm.at[wid, pl.ds(CH * W1 * win, CH * W1)])
        return carry0

    lax.fori_loop(0, NWIN1, window_body, 0)
    plsc.subcore_barrier()

    @pl.when(sub < 10)
    def _writeback():
        pltpu.sync_copy(den_sh.at[pl.ds(TSL * sub, TSL)],
                        den_part_hbm.at[core, pl.ds(TSL * sub, TSL)])


def _pass1(s2, ei4, zeros):
    mesh = plsc.VectorSubcoreMesh(core_axis_name="c", subcore_axis_name="s")
    return pl.kernel(
        _pass1_body,
        out_type=[
            jax.ShapeDtypeStruct((2, N, 8), jnp.float32),
            jax.ShapeDtypeStruct((NW, EPW), jnp.float32),
        ],
        mesh=mesh,
        scratch_types=[
            pltpu.VMEM((W1, CH), jnp.int32),
            pltpu.VMEM((W1, CH), jnp.int32),
            pltpu.VMEM((N,), jnp.float32),
            pltpu.VMEM((N,), jnp.float32),
            pltpu.VMEM((CH, 8), jnp.float32),
            pltpu.VMEM((CH, 8), jnp.float32),
            pltpu.VMEM((W1 * CH,), jnp.float32),
            pltpu.VMEM_SHARED((N, 8), jnp.float32),
            pltpu.SemaphoreType.DMA,
            pltpu.SemaphoreType.DMA,
        ],
        compiler_params=_SC_PARAMS,
    )(s2, ei4, zeros)


def _pass2_body(xw16_hbm, ei4_hbm, w_hbm, zeros_hbm, out_part_hbm,
                src_idx_v, dst_idx_v, wlin_v,
                rows0_v, rows1_v, rows2_v, rows3_v, stg0_v, stg1_v, out_sh,
                gsem0, gsem1, gsem2, gsem3, ssem0, ssem1):
    core = lax.axis_index("c")
    sub = lax.axis_index("s")
    wid = core * 16 + sub
    bufs = [rows0_v, rows1_v, rows2_v, rows3_v]
    gsems = [gsem0, gsem1, gsem2, gsem3]
    stgs = [stg0_v, stg1_v]
    ssems = [ssem0, ssem1]

    @pl.when(sub < 10)
    def _zero():
        pltpu.sync_copy(zeros_hbm.at[pl.ds(TSL * sub, TSL)],
                        out_sh.at[pl.ds(TSL * sub, TSL)])
    plsc.subcore_barrier()

    iota16 = lax.iota(jnp.int32, 16)

    def fire_gather(c, i):
        pltpu.async_copy(xw16_hbm.at[src_idx_v.at[c]], bufs[i], gsems[i])

    def wait_gather(i):
        pltpu.make_async_copy(xw16_hbm.at[src_idx_v.at[0]], bufs[i],
                              gsems[i]).wait()

    def fire_scatter(c, p):
        pltpu.async_copy(stgs[p], out_sh.at[dst_idx_v.at[c]], ssems[p],
                         add=True)

    def wait_scatter(p):
        pltpu.make_async_copy(stgs[p], out_sh.at[dst_idx_v.at[0]],
                              ssems[p]).wait()

    def scale(i, p, wbase):
        rows = bufs[i]
        stg = stgs[p]

        @plsc.parallel_loop(0, CH, 1, unroll=8)
        def row_body(r):
            wsp = plsc.load_gather(wlin_v, [jnp.full((16,), wbase + r,
                                                     jnp.int32)])
            rsplat = jnp.full((16,), r, jnp.int32)
            for g in range(C // 32):
                v = rows[r, pl.ds(32 * g, 32)]
                a, b = plsc.unpack(v, format=plsc.PackFormat.INTERLEAVED)
                plsc.store_scatter(stg, [rsplat, 32 * g + 2 * iota16],
                                   a * wsp)
                plsc.store_scatter(stg, [rsplat, 32 * g + 1 + 2 * iota16],
                                   b * wsp)

    def window_body(win, carry0):
        @pl.when(win > 0)
        def _drain():
            wait_scatter(0)
            wait_scatter(1)
        pltpu.sync_copy(ei4_hbm.at[0, wid, pl.ds(W2 * win, W2)], src_idx_v)
        pltpu.sync_copy(ei4_hbm.at[1, wid, pl.ds(W2 * win, W2)], dst_idx_v)
        pltpu.sync_copy(w_hbm.at[wid, pl.ds(CH * W2 * win, CH * W2)], wlin_v)

        for i in range(3):
            fire_gather(i, i)

        def quad_body(q, carry):
            for i in range(4):
                c = 4 * q + i
                p = i % 2

                @pl.when(c < W2 - 3)
                def _fg(c=c, i=i):
                    fire_gather(c + 3, (i + 3) % 4)

                wait_gather(i)

                @pl.when(c >= 2)
                def _ws(p=p):
                    wait_scatter(p)

                scale(i, p, CH * c)
                fire_scatter(c, p)
            return carry

        lax.fori_loop(0, W2 // 4, quad_body, 0)
        return carry0

    lax.fori_loop(0, NWIN2, window_body, 0)

    wait_scatter(0)
    wait_scatter(1)
    pltpu.sync_copy(ei4_hbm.at[0, wid, pl.ds(NWIN2 * W2, TAIL2)],
                    src_idx_v.at[pl.ds(0, TAIL2)])
    pltpu.sync_copy(ei4_hbm.at[1, wid, pl.ds(NWIN2 * W2, TAIL2)],
                    dst_idx_v.at[pl.ds(0, TAIL2)])
    pltpu.sync_copy(w_hbm.at[wid, pl.ds(CH * NWIN2 * W2, CH * TAIL2)],
                    wlin_v.at[pl.ds(0, CH * TAIL2)])
    for i in range(3):
        fire_gather(i, i)
    fire_gather(3, 3)
    wait_gather(0)
    scale(0, 0, 0)
    fire_scatter(0, 0)
    fire_gather(4, 0)
    wait_gather(1)
    scale(1, 1, CH)
    fire_scatter(1, 1)
    for c in range(2, TAIL2):
        i = c % 4
        p = c % 2
        wait_gather(i)
        wait_scatter(p)
        scale(i, p, CH * c)
        fire_scatter(c, p)
    wait_scatter(1)
    wait_scatter(0)

    plsc.subcore_barrier()

    @pl.when(sub < 10)
    def _writeback():
        pltpu.sync_copy(out_sh.at[pl.ds(TSL * sub, TSL)],
                        out_part_hbm.at[core, pl.ds(TSL * sub, TSL)])


def _pass2(xw16, ei4, w, zeros):
    mesh = plsc.VectorSubcoreMesh(core_axis_name="c", subcore_axis_name="s")
    return pl.kernel(
        _pass2_body,
        out_type=jax.ShapeDtypeStruct((2, N, C), jnp.float32),
        mesh=mesh,
        scratch_types=[
            pltpu.VMEM((W2, CH), jnp.int32),
            pltpu.VMEM((W2, CH), jnp.int32),
            pltpu.VMEM((W2 * CH,), jnp.float32),
            pltpu.VMEM((CH, C), jnp.bfloat16),
            pltpu.VMEM((CH, C), jnp.bfloat16),
            pltpu.VMEM((CH, C), jnp.bfloat16),
            pltpu.VMEM((CH, C), jnp.bfloat16),
            pltpu.VMEM((CH, C), jnp.float32),
            pltpu.VMEM((CH, C), jnp.float32),
            pltpu.VMEM_SHARED((N, C), jnp.float32),
            pltpu.SemaphoreType.DMA,
            pltpu.SemaphoreType.DMA,
            pltpu.SemaphoreType.DMA,
            pltpu.SemaphoreType.DMA,
            pltpu.SemaphoreType.DMA,
            pltpu.SemaphoreType.DMA,
        ],
        compiler_params=_SC_PARAMS,
    )(xw16, ei4, w, zeros)


def _stage_c_body(op_ref, den_ref, bias_ref, out_ref):
    p = op_ref[0] + op_ref[1]
    d = den_ref[0, :, 0] + den_ref[1, :, 0]
    out_ref[...] = p / (d[:, None] + 1e-16) + bias_ref[...][None, :]


def _stage_c(out_part, den_part, bias):
    blk = 1000
    return pl.pallas_call(
        _stage_c_body,
        grid=(N // blk,),
        in_specs=[
            pl.BlockSpec((2, blk, C), lambda i: (0, i, 0)),
            pl.BlockSpec((2, blk, 8), lambda i: (0, i, 0)),
            pl.BlockSpec((C,), lambda i: (0,)),
        ],
        out_specs=pl.BlockSpec((blk, C), lambda i: (i, 0)),
        out_shape=jax.ShapeDtypeStruct((N, C), jnp.float32),
    )(out_part, den_part, bias)


@jax.jit
def kernel(x, edge_index, mask, weight, att, bias):
    del mask
    att2 = att.reshape(2, C)
    ei4 = edge_index.reshape(2, NW, NCH, CH)
    zeros = jnp.zeros((N, C), jnp.float32)
    xw, s2 = _stage_a(x, weight, att2)
    den_part, w = _pass1(s2, ei4, zeros)
    out_part = _pass2(xw, ei4, w, zeros)
    return _stage_c(out_part, den_part, bias)


# speedup vs baseline: 1.0163x; 1.0163x over previous
import jax
import jax.numpy as jnp
from jax import lax
from jax.experimental import pallas as pl
from jax.experimental.pallas import tpu as pltpu, tpu_sc as plsc

N = 10000
E = 320000
C = 128
NEG_SLOPE = 0.2

NW = 32
EPW = E // NW
CH = 80
NCH = EPW // CH
W1 = 25
NWIN1 = NCH // W1
W2 = 20
NWIN2 = 6
TAIL2 = NCH - NWIN2 * W2
TSL = 1000

_SC_PARAMS = pltpu.CompilerParams(
    use_tc_tiling_on_sc=False, needs_layout_passes=False)


def _stage_a_body(x_ref, w_ref, att2_ref, xw16_ref, s2_ref):
    xwb = jnp.dot(x_ref[...], w_ref[...], preferred_element_type=jnp.float32)
    xw16_ref[...] = xwb.astype(jnp.bfloat16)
    s2_ref[...] = lax.dot_general(
        att2_ref[...], xwb, (((1,), (1,)), ((), ())),
        preferred_element_type=jnp.float32)


def _stage_a(x, weight, att2):
    return pl.pallas_call(
        _stage_a_body,
        out_shape=[
            jax.ShapeDtypeStruct((N, C), jnp.bfloat16),
            jax.ShapeDtypeStruct((2, N), jnp.float32),
        ],
    )(x, weight, att2)


def _pass1_body(s2_hbm, ei4_hbm, zeros_hbm, den_part_hbm, w_hbm,
                src_idx_v, dst_idx_v, sdst_v, ssrc_v,
                w8a_v, w8b_v, w8c_v, w8d_v, wlin_v,
                den_sh, wsema, wsemb, wsemc, wsemd):
    core = lax.axis_index("c")
    sub = lax.axis_index("s")
    wid = core * 16 + sub
    w8s = [w8a_v, w8b_v, w8c_v, w8d_v]
    wsems = [wsema, wsemb, wsemc, wsemd]

    pltpu.sync_copy(s2_hbm.at[0], sdst_v)
    pltpu.sync_copy(s2_hbm.at[1], ssrc_v)
    for w8 in w8s:
        pltpu.sync_copy(zeros_hbm.at[pl.ds(0, CH), pl.ds(0, 8)], w8)

    @pl.when(sub < 10)
    def _zero():
        pltpu.sync_copy(zeros_hbm.at[pl.ds(TSL * sub, TSL), pl.ds(0, 8)],
                        den_sh.at[pl.ds(TSL * sub, TSL)])
    plsc.subcore_barrier()

    zeros16 = jnp.zeros((16,), jnp.int32)
    iota16 = lax.iota(jnp.int32, 16)

    def compute_w(j, w8):
        for k in range(CH // 16):
            di = dst_idx_v[j, pl.ds(16 * k, 16)]
            si = src_idx_v[j, pl.ds(16 * k, 16)]
            a = (plsc.load_gather(sdst_v, [di])
                 + plsc.load_gather(ssrc_v, [si]))
            a = jnp.maximum(a, NEG_SLOPE * a)
            w = jnp.exp(a)
            plsc.store_scatter(w8, [iota16 + 16 * k, zeros16], w)
            wlin_v[pl.ds(CH * j + 16 * k, 16)] = w

    def window_body(win, carry0):
        pltpu.sync_copy(ei4_hbm.at[0, wid, pl.ds(W1 * win, W1)], src_idx_v)
        pltpu.sync_copy(ei4_hbm.at[1, wid, pl.ds(W1 * win, W1)], dst_idx_v)

        def quad_body(jq, carry):
            descs = []
            for i in range(4):
                j = 4 * jq + i
                compute_w(j, w8s[i])
                descs.append(pltpu.async_copy(
                    w8s[i], den_sh.at[dst_idx_v.at[j]], wsems[i], add=True))
            for d in descs:
                d.wait()
            return carry

        lax.fori_loop(0, W1 // 4, quad_body, 0)
        compute_w(W1 - 1, w8a_v)
        pltpu.async_copy(w8a_v, den_sh.at[dst_idx_v.at[W1 - 1]], wsema,
                         add=True).wait()
        pltpu.sync_copy(wlin_v, w_hbm.at[wid, pl.ds(CH * W1 * win, CH * W1)])
        return carry0

    lax.fori_loop(0, NWIN1, window_body, 0)
    plsc.subcore_barrier()

    @pl.when(sub < 10)
    def _writeback():
        pltpu.sync_copy(den_sh.at[pl.ds(TSL * sub, TSL)],
                        den_part_hbm.at[core, pl.ds(TSL * sub, TSL)])


def _pass1(s2, ei4, zeros):
    mesh = plsc.VectorSubcoreMesh(core_axis_name="c", subcore_axis_name="s")
    return pl.kernel(
        _pass1_body,
        out_type=[
            jax.ShapeDtypeStruct((2, N, 8), jnp.float32),
            jax.ShapeDtypeStruct((NW, EPW), jnp.float32),
        ],
        mesh=mesh,
        scratch_types=[
            pltpu.VMEM((W1, CH), jnp.int32),
            pltpu.VMEM((W1, CH), jnp.int32),
            pltpu.VMEM((N,), jnp.float32),
            pltpu.VMEM((N,), jnp.float32),
            pltpu.VMEM((CH, 8), jnp.float32),
            pltpu.VMEM((CH, 8), jnp.float32),
            pltpu.VMEM((CH, 8), jnp.float32),
            pltpu.VMEM((CH, 8), jnp.float32),
            pltpu.VMEM((W1 * CH,), jnp.float32),
            pltpu.VMEM_SHARED((N, 8), jnp.float32),
            pltpu.SemaphoreType.DMA,
            pltpu.SemaphoreType.DMA,
            pltpu.SemaphoreType.DMA,
            pltpu.SemaphoreType.DMA,
        ],
        compiler_params=_SC_PARAMS,
    )(s2, ei4, zeros)


def _pass2_body(xw16_hbm, ei4_hbm, w_hbm, zeros_hbm, out_part_hbm,
                src_idx_v, dst_idx_v, wlin_v,
                rows0_v, rows1_v, rows2_v, rows3_v, stg0_v, stg1_v, out_sh,
                gsem0, gsem1, gsem2, gsem3, ssem0, ssem1):
    core = lax.axis_index("c")
    sub = lax.axis_index("s")
    wid = core * 16 + sub
    bufs = [rows0_v, rows1_v, rows2_v, rows3_v]
    gsems = [gsem0, gsem1, gsem2, gsem3]
    stgs = [stg0_v, stg1_v]
    ssems = [ssem0, ssem1]

    @pl.when(sub < 10)
    def _zero():
        pltpu.sync_copy(zeros_hbm.at[pl.ds(TSL * sub, TSL)],
                        out_sh.at[pl.ds(TSL * sub, TSL)])
    plsc.subcore_barrier()

    iota16 = lax.iota(jnp.int32, 16)

    def fire_gather(c, i):
        pltpu.async_copy(xw16_hbm.at[src_idx_v.at[c]], bufs[i], gsems[i])

    def wait_gather(i):
        pltpu.make_async_copy(xw16_hbm.at[src_idx_v.at[0]], bufs[i],
                              gsems[i]).wait()

    def fire_scatter(c, p):
        pltpu.async_copy(stgs[p], out_sh.at[dst_idx_v.at[c]], ssems[p],
                         add=True)

    def wait_scatter(p):
        pltpu.make_async_copy(stgs[p], out_sh.at[dst_idx_v.at[0]],
                              ssems[p]).wait()

    def scale(i, p, wbase):
        rows = bufs[i]
        stg = stgs[p]

        @plsc.parallel_loop(0, CH, 1, unroll=4)
        def row_body(r):
            wsp = plsc.load_gather(wlin_v, [jnp.full((16,), wbase + r,
                                                     jnp.int32)])
            rsplat = jnp.full((16,), r, jnp.int32)
            for g in range(C // 32):
                v = rows[r, pl.ds(32 * g, 32)]
                a, b = plsc.unpack(v, format=plsc.PackFormat.INTERLEAVED)
                plsc.store_scatter(stg, [rsplat, 32 * g + 2 * iota16],
                                   a * wsp)
                plsc.store_scatter(stg, [rsplat, 32 * g + 1 + 2 * iota16],
                                   b * wsp)

    def window_body(win, carry0):
        @pl.when(win > 0)
        def _drain():
            wait_scatter(0)
            wait_scatter(1)
        pltpu.sync_copy(ei4_hbm.at[0, wid, pl.ds(W2 * win, W2)], src_idx_v)
        pltpu.sync_copy(ei4_hbm.at[1, wid, pl.ds(W2 * win, W2)], dst_idx_v)
        pltpu.sync_copy(w_hbm.at[wid, pl.ds(CH * W2 * win, CH * W2)], wlin_v)

        for i in range(3):
            fire_gather(i, i)

        def quad_body(q, carry):
            for i in range(4):
                c = 4 * q + i
                p = i % 2

                @pl.when(c < W2 - 3)
                def _fg(c=c, i=i):
                    fire_gather(c + 3, (i + 3) % 4)

                wait_gather(i)

                @pl.when(c >= 2)
                def _ws(p=p):
                    wait_scatter(p)

                scale(i, p, CH * c)
                fire_scatter(c, p)
            return carry

        lax.fori_loop(0, W2 // 4, quad_body, 0)
        return carry0

    lax.fori_loop(0, NWIN2, window_body, 0)

    wait_scatter(0)
    wait_scatter(1)
    pltpu.sync_copy(ei4_hbm.at[0, wid, pl.ds(NWIN2 * W2, TAIL2)],
                    src_idx_v.at[pl.ds(0, TAIL2)])
    pltpu.sync_copy(ei4_hbm.at[1, wid, pl.ds(NWIN2 * W2, TAIL2)],
                    dst_idx_v.at[pl.ds(0, TAIL2)])
    pltpu.sync_copy(w_hbm.at[wid, pl.ds(CH * NWIN2 * W2, CH * TAIL2)],
                    wlin_v.at[pl.ds(0, CH * TAIL2)])
    for i in range(3):
        fire_gather(i, i)
    fire_gather(3, 3)
    wait_gather(0)
    scale(0, 0, 0)
    fire_scatter(0, 0)
    fire_gather(4, 0)
    wait_gather(1)
    scale(1, 1, CH)
    fire_scatter(1, 1)
    for c in range(2, TAIL2):
        i = c % 4
        p = c % 2
        wait_gather(i)
        wait_scatter(p)
        scale(i, p, CH * c)
        fire_scatter(c, p)
    wait_scatter(1)
    wait_scatter(0)

    plsc.subcore_barrier()

    @pl.when(sub < 10)
    def _writeback():
        pltpu.sync_copy(out_sh.at[pl.ds(TSL * sub, TSL)],
                        out_part_hbm.at[core, pl.ds(TSL * sub, TSL)])


def _pass2(xw16, ei4, w, zeros):
    mesh = plsc.VectorSubcoreMesh(core_axis_name="c", subcore_axis_name="s")
    return pl.kernel(
        _pass2_body,
        out_type=jax.ShapeDtypeStruct((2, N, C), jnp.float32),
        mesh=mesh,
        scratch_types=[
            pltpu.VMEM((W2, CH), jnp.int32),
            pltpu.VMEM((W2, CH), jnp.int32),
            pltpu.VMEM((W2 * CH,), jnp.float32),
            pltpu.VMEM((CH, C), jnp.bfloat16),
            pltpu.VMEM((CH, C), jnp.bfloat16),
            pltpu.VMEM((CH, C), jnp.bfloat16),
            pltpu.VMEM((CH, C), jnp.bfloat16),
            pltpu.VMEM((CH, C), jnp.float32),
            pltpu.VMEM((CH, C), jnp.float32),
            pltpu.VMEM_SHARED((N, C), jnp.float32),
            pltpu.SemaphoreType.DMA,
            pltpu.SemaphoreType.DMA,
            pltpu.SemaphoreType.DMA,
            pltpu.SemaphoreType.DMA,
            pltpu.SemaphoreType.DMA,
            pltpu.SemaphoreType.DMA,
        ],
        compiler_params=_SC_PARAMS,
    )(xw16, ei4, w, zeros)


def _stage_c_body(op_ref, den_ref, bias_ref, out_ref):
    p = op_ref[0] + op_ref[1]
    d = den_ref[0, :, 0] + den_ref[1, :, 0]
    out_ref[...] = p / (d[:, None] + 1e-16) + bias_ref[...][None, :]


def _stage_c(out_part, den_part, bias):
    blk = 1000
    return pl.pallas_call(
        _stage_c_body,
        grid=(N // blk,),
        in_specs=[
            pl.BlockSpec((2, blk, C), lambda i: (0, i, 0)),
            pl.BlockSpec((2, blk, 8), lambda i: (0, i, 0)),
            pl.BlockSpec((C,), lambda i: (0,)),
        ],
        out_specs=pl.BlockSpec((blk, C), lambda i: (i, 0)),
        out_shape=jax.ShapeDtypeStruct((N, C), jnp.float32),
    )(out_part, den_part, bias)


@jax.jit
def kernel(x, edge_index, mask, weight, att, bias):
    del mask
    att2 = att.reshape(2, C)
    ei4 = edge_index.reshape(2, NW, NCH, CH)
    zeros = jnp.zeros((N, C), jnp.float32)
    xw, s2 = _stage_a(x, weight, att2)
    den_part, w = _pass1(s2, ei4, zeros)
    out_part = _pass2(xw, ei4, w, zeros)
    return _stage_c(out_part, den_part, bias)


# speedup vs baseline: 1.0210x; 1.0046x over previous
import jax
import jax.numpy as jnp
from jax import lax
from jax.experimental import pallas as pl
from jax.experimental.pallas import tpu as pltpu, tpu_sc as plsc

N = 10000
E = 320000
C = 128
NEG_SLOPE = 0.2

NW = 32
EPW = E // NW
CH = 80
NCH = EPW // CH
W1 = 25
NWIN1 = NCH // W1
W2 = 20
NWIN2 = 6
TAIL2 = NCH - NWIN2 * W2
TSL = 1000

_SC_PARAMS = pltpu.CompilerParams(
    use_tc_tiling_on_sc=False, needs_layout_passes=False)


def _stage_a_body(x_ref, w_ref, att2_ref, xw16_ref, s2_ref):
    xwb = jnp.dot(x_ref[...], w_ref[...], preferred_element_type=jnp.float32)
    xw16_ref[...] = xwb.astype(jnp.bfloat16)
    s2_ref[...] = lax.dot_general(
        att2_ref[...], xwb, (((1,), (1,)), ((), ())),
        preferred_element_type=jnp.float32)


def _stage_a(x, weight, att2):
    return pl.pallas_call(
        _stage_a_body,
        out_shape=[
            jax.ShapeDtypeStruct((N, C), jnp.bfloat16),
            jax.ShapeDtypeStruct((2, N), jnp.float32),
        ],
    )(x, weight, att2)


def _pass1_body(s2_hbm, ei4_hbm, zeros_hbm, den_part_hbm, w_hbm,
                src_idx_v, dst_idx_v, sdst_v, ssrc_v,
                w8a_v, w8b_v, w8c_v, w8d_v, wlin_v,
                den_sh, wsema, wsemb, wsemc, wsemd):
    core = lax.axis_index("c")
    sub = lax.axis_index("s")
    wid = core * 16 + sub
    w8s = [w8a_v, w8b_v, w8c_v, w8d_v]
    wsems = [wsema, wsemb, wsemc, wsemd]

    pltpu.sync_copy(s2_hbm.at[0], sdst_v)
    pltpu.sync_copy(s2_hbm.at[1], ssrc_v)
    for w8 in w8s:
        pltpu.sync_copy(zeros_hbm.at[pl.ds(0, CH), pl.ds(0, 8)], w8)

    @pl.when(sub < 10)
    def _zero():
        pltpu.sync_copy(zeros_hbm.at[pl.ds(TSL * sub, TSL), pl.ds(0, 8)],
                        den_sh.at[pl.ds(TSL * sub, TSL)])
    plsc.subcore_barrier()

    zeros16 = jnp.zeros((16,), jnp.int32)
    iota16 = lax.iota(jnp.int32, 16)

    def compute_w(j, w8):
        for k in range(CH // 16):
            di = dst_idx_v[j, pl.ds(16 * k, 16)]
            si = src_idx_v[j, pl.ds(16 * k, 16)]
            a = (plsc.load_gather(sdst_v, [di])
                 + plsc.load_gather(ssrc_v, [si]))
            a = jnp.maximum(a, NEG_SLOPE * a)
            w = jnp.exp(a)
            plsc.store_scatter(w8, [iota16 + 16 * k, zeros16], w)
            wlin_v[pl.ds(CH * j + 16 * k, 16)] = w

    def window_body(win, carry0):
        pltpu.sync_copy(ei4_hbm.at[0, wid, pl.ds(W1 * win, W1)], src_idx_v)
        pltpu.sync_copy(ei4_hbm.at[1, wid, pl.ds(W1 * win, W1)], dst_idx_v)

        def quad_body(jq, carry):
            descs = []
            for i in range(4):
                j = 4 * jq + i
                compute_w(j, w8s[i])
                descs.append(pltpu.async_copy(
                    w8s[i], den_sh.at[dst_idx_v.at[j]], wsems[i], add=True))
            for d in descs:
                d.wait()
            return carry

        lax.fori_loop(0, W1 // 4, quad_body, 0)
        compute_w(W1 - 1, w8a_v)
        pltpu.async_copy(w8a_v, den_sh.at[dst_idx_v.at[W1 - 1]], wsema,
                         add=True).wait()
        pltpu.sync_copy(wlin_v, w_hbm.at[wid, pl.ds(CH * W1 * win, CH * W1)])
        return carry0

    lax.fori_loop(0, NWIN1, window_body, 0)
    plsc.subcore_barrier()

    @pl.when(sub < 10)
    def _writeback():
        pltpu.sync_copy(den_sh.at[pl.ds(TSL * sub, TSL)],
                        den_part_hbm.at[core, pl.ds(TSL * sub, TSL)])


def _pass1(s2, ei4, zeros):
    mesh = plsc.VectorSubcoreMesh(core_axis_name="c", subcore_axis_name="s")
    return pl.kernel(
        _pass1_body,
        out_type=[
            jax.ShapeDtypeStruct((2, N, 8), jnp.float32),
            jax.ShapeDtypeStruct((NW, EPW), jnp.float32),
        ],
        mesh=mesh,
        scratch_types=[
            pltpu.VMEM((W1, CH), jnp.int32),
            pltpu.VMEM((W1, CH), jnp.int32),
            pltpu.VMEM((N,), jnp.float32),
            pltpu.VMEM((N,), jnp.float32),
            pltpu.VMEM((CH, 8), jnp.float32),
            pltpu.VMEM((CH, 8), jnp.float32),
            pltpu.VMEM((CH, 8), jnp.float32),
            pltpu.VMEM((CH, 8), jnp.float32),
            pltpu.VMEM((W1 * CH,), jnp.float32),
            pltpu.VMEM_SHARED((N, 8), jnp.float32),
            pltpu.SemaphoreType.DMA,
            pltpu.SemaphoreType.DMA,
            pltpu.SemaphoreType.DMA,
            pltpu.SemaphoreType.DMA,
        ],
        compiler_params=_SC_PARAMS,
    )(s2, ei4, zeros)


def _pass2_body(xw16_hbm, ei4_hbm, w_hbm, zeros_hbm, out_part_hbm,
                src_idx_v, dst_idx_v, wlin_v,
                rows0_v, rows1_v, rows2_v, rows3_v, stg0_v, stg1_v, out_sh,
                gsem0, gsem1, gsem2, gsem3, ssem0, ssem1):
    core = lax.axis_index("c")
    sub = lax.axis_index("s")
    wid = core * 16 + sub
    bufs = [rows0_v, rows1_v, rows2_v, rows3_v]
    gsems = [gsem0, gsem1, gsem2, gsem3]
    stgs = [stg0_v, stg1_v]
    ssems = [ssem0, ssem1]

    @pl.when(sub < 10)
    def _zero():
        pltpu.sync_copy(zeros_hbm.at[pl.ds(TSL * sub, TSL)],
                        out_sh.at[pl.ds(TSL * sub, TSL)])
    plsc.subcore_barrier()

    iota16 = lax.iota(jnp.int32, 16)

    def fire_gather(c, i):
        pltpu.async_copy(xw16_hbm.at[src_idx_v.at[c]], bufs[i], gsems[i])

    def wait_gather(i):
        pltpu.make_async_copy(xw16_hbm.at[src_idx_v.at[0]], bufs[i],
                              gsems[i]).wait()

    def fire_scatter(c, p):
        pltpu.async_copy(stgs[p], out_sh.at[dst_idx_v.at[c]], ssems[p],
                         add=True)

    def wait_scatter(p):
        pltpu.make_async_copy(stgs[p], out_sh.at[dst_idx_v.at[0]],
                              ssems[p]).wait()

    def scale(i, p, wbase):
        rows = bufs[i]
        stg = stgs[p]

        @plsc.parallel_loop(0, CH, 1, unroll=2)
        def row_body(r):
            wsp = plsc.load_gather(wlin_v, [jnp.full((16,), wbase + r,
                                                     jnp.int32)])
            rsplat = jnp.full((16,), r, jnp.int32)
            for g in range(C // 32):
                v = rows[r, pl.ds(32 * g, 32)]
                a, b = plsc.unpack(v, format=plsc.PackFormat.INTERLEAVED)
                plsc.store_scatter(stg, [rsplat, 32 * g + 2 * iota16],
                                   a * wsp)
                plsc.store_scatter(stg, [rsplat, 32 * g + 1 + 2 * iota16],
                                   b * wsp)

    def window_body(win, carry0):
        @pl.when(win > 0)
        def _drain():
            wait_scatter(0)
            wait_scatter(1)
        pltpu.sync_copy(ei4_hbm.at[0, wid, pl.ds(W2 * win, W2)], src_idx_v)
        pltpu.sync_copy(ei4_hbm.at[1, wid, pl.ds(W2 * win, W2)], dst_idx_v)
        pltpu.sync_copy(w_hbm.at[wid, pl.ds(CH * W2 * win, CH * W2)], wlin_v)

        for i in range(3):
            fire_gather(i, i)

        def quad_body(q, carry):
            for i in range(4):
                c = 4 * q + i
                p = i % 2

                @pl.when(c < W2 - 3)
                def _fg(c=c, i=i):
                    fire_gather(c + 3, (i + 3) % 4)

                wait_gather(i)

                @pl.when(c >= 2)
                def _ws(p=p):
                    wait_scatter(p)

                scale(i, p, CH * c)
                fire_scatter(c, p)
            return carry

        lax.fori_loop(0, W2 // 4, quad_body, 0)
        return carry0

    lax.fori_loop(0, NWIN2, window_body, 0)

    wait_scatter(0)
    wait_scatter(1)
    pltpu.sync_copy(ei4_hbm.at[0, wid, pl.ds(NWIN2 * W2, TAIL2)],
                    src_idx_v.at[pl.ds(0, TAIL2)])
    pltpu.sync_copy(ei4_hbm.at[1, wid, pl.ds(NWIN2 * W2, TAIL2)],
                    dst_idx_v.at[pl.ds(0, TAIL2)])
    pltpu.sync_copy(w_hbm.at[wid, pl.ds(CH * NWIN2 * W2, CH * TAIL2)],
                    wlin_v.at[pl.ds(0, CH * TAIL2)])
    for i in range(3):
        fire_gather(i, i)
    fire_gather(3, 3)
    wait_gather(0)
    scale(0, 0, 0)
    fire_scatter(0, 0)
    fire_gather(4, 0)
    wait_gather(1)
    scale(1, 1, CH)
    fire_scatter(1, 1)
    for c in range(2, TAIL2):
        i = c % 4
        p = c % 2
        wait_gather(i)
        wait_scatter(p)
        scale(i, p, CH * c)
        fire_scatter(c, p)
    wait_scatter(1)
    wait_scatter(0)

    plsc.subcore_barrier()

    @pl.when(sub < 10)
    def _writeback():
        pltpu.sync_copy(out_sh.at[pl.ds(TSL * sub, TSL)],
                        out_part_hbm.at[core, pl.ds(TSL * sub, TSL)])


def _pass2(xw16, ei4, w, zeros):
    mesh = plsc.VectorSubcoreMesh(core_axis_name="c", subcore_axis_name="s")
    return pl.kernel(
        _pass2_body,
        out_type=jax.ShapeDtypeStruct((2, N, C), jnp.float32),
        mesh=mesh,
        scratch_types=[
            pltpu.VMEM((W2, CH), jnp.int32),
            pltpu.VMEM((W2, CH), jnp.int32),
            pltpu.VMEM((W2 * CH,), jnp.float32),
            pltpu.VMEM((CH, C), jnp.bfloat16),
            pltpu.VMEM((CH, C), jnp.bfloat16),
            pltpu.VMEM((CH, C), jnp.bfloat16),
            pltpu.VMEM((CH, C), jnp.bfloat16),
            pltpu.VMEM((CH, C), jnp.float32),
            pltpu.VMEM((CH, C), jnp.float32),
            pltpu.VMEM_SHARED((N, C), jnp.float32),
            pltpu.SemaphoreType.DMA,
            pltpu.SemaphoreType.DMA,
            pltpu.SemaphoreType.DMA,
            pltpu.SemaphoreType.DMA,
            pltpu.SemaphoreType.DMA,
            pltpu.SemaphoreType.DMA,
        ],
        compiler_params=_SC_PARAMS,
    )(xw16, ei4, w, zeros)


def _stage_c_body(op_ref, den_ref, bias_ref, out_ref):
    p = op_ref[0] + op_ref[1]
    d = den_ref[0, :, 0] + den_ref[1, :, 0]
    out_ref[...] = p / (d[:, None] + 1e-16) + bias_ref[...][None, :]


def _stage_c(out_part, den_part, bias):
    blk = 1000
    return pl.pallas_call(
        _stage_c_body,
        grid=(N // blk,),
        in_specs=[
            pl.BlockSpec((2, blk, C), lambda i: (0, i, 0)),
            pl.BlockSpec((2, blk, 8), lambda i: (0, i, 0)),
            pl.BlockSpec((C,), lambda i: (0,)),
        ],
        out_specs=pl.BlockSpec((blk, C), lambda i: (i, 0)),
        out_shape=jax.ShapeDtypeStruct((N, C), jnp.float32),
    )(out_part, den_part, bias)


@jax.jit
def kernel(x, edge_index, mask, weight, att, bias):
    del mask
    att2 = att.reshape(2, C)
    ei4 = edge_index.reshape(2, NW, NCH, CH)
    zeros = jnp.zeros((N, C), jnp.float32)
    xw, s2 = _stage_a(x, weight, att2)
    den_part, w = _pass1(s2, ei4, zeros)
    out_part = _pass2(xw, ei4, w, zeros)
    return _stage_c(out_part, den_part, bias)
